# baseline (speedup 1.0000x reference)
import jax
import jax.numpy as jnp
from jax.experimental import pallas as pl


_L = 1024


def _body(x_ref, o_ref):
    b = pl.program_id(0)
    k = pl.program_id(1)
    part = jnp.max(x_ref[...], axis=0)

    @pl.when(k == 0)
    def _():
        o_ref[b, :] = part

    @pl.when(k != 0)
    def _():
        o_ref[b, :] = jnp.maximum(o_ref[b, :], part)


def kernel(x, lengths):
    del lengths
    T, D = x.shape
    B = T // _L
    K = 8
    C = _L // K

    out = pl.pallas_call(
        _body,
        grid=(B, K),
        in_specs=[pl.BlockSpec((C, D), lambda b, k: (b * K + k, 0))],
        out_specs=pl.BlockSpec((B, D), lambda b, k: (0, 0)),
        out_shape=jax.ShapeDtypeStruct((B, D), x.dtype),
    )(x)
    return (out, None)


# speedup vs baseline: 3.1024x; 3.1024x over previous
import jax
import jax.numpy as jnp
from jax.experimental import pallas as pl


_L = 1024


def _body(x_ref, o_ref):
    k = pl.program_id(0)
    part = jnp.max(x_ref[...], axis=1)

    @pl.when(k == 0)
    def _():
        o_ref[...] = part

    @pl.when(k != 0)
    def _():
        o_ref[...] = jnp.maximum(o_ref[...], part)


def kernel(x, lengths):
    del lengths
    T, D = x.shape
    B = T // _L
    K = 8
    C = _L // K

    xr = x.reshape(B, _L, D)
    out = pl.pallas_call(
        _body,
        grid=(K,),
        in_specs=[pl.BlockSpec((B, C, D), lambda k: (0, k, 0))],
        out_specs=pl.BlockSpec((B, D), lambda k: (0, 0)),
        out_shape=jax.ShapeDtypeStruct((B, D), x.dtype),
    )(xr)
    return (out, None)
